# baseline (speedup 1.0000x reference)
import jax
import jax.numpy as jnp
from jax.experimental import pallas as pl
from jax.experimental.pallas import tpu as pltpu


_B = 2000


def _stage1(h_ref, y_ref, a_ref, s_ref):
    x = h_ref[...]
    yv = y_ref[...]
    b, c = x.shape
    m = jnp.max(x, axis=-1, keepdims=True)
    e = jnp.exp(x - m)
    col = jax.lax.broadcasted_iota(jnp.int32, (b, c), 1)
    g = jnp.where(col == yv[0, 0][:, None], x, 0.0)
    ones = jnp.ones((c, 8), jnp.float32)
    s8 = jax.lax.dot_general(e, ones, (((1,), (0,)), ((), ())),
                             preferred_element_type=jnp.float32)
    g8 = jax.lax.dot_general(g, ones, (((1,), (0,)), ((), ())),
                             preferred_element_type=jnp.float32)
    a_ref[...] = g8[:, :1] - m
    s_ref[...] = s8[:, :1]


def _stage2(scale_ref, a_ref, s_ref, o_ref):
    o_ref[...] = (a_ref[...] - jnp.log(s_ref[...])) * scale_ref[0]


def kernel(h, y, num_edges):
    n, c = h.shape
    nb = n // _B
    scale = (num_edges / n).astype(jnp.float32).reshape(1)
    y3 = y.astype(jnp.int32).reshape(nb, 1, _B)
    a, s = pl.pallas_call(
        _stage1,
        grid=(nb,),
        in_specs=[
            pl.BlockSpec((_B, c), lambda i: (i, 0)),
            pl.BlockSpec((1, 1, _B), lambda i: (i, 0, 0)),
        ],
        out_specs=[
            pl.BlockSpec((_B, 1), lambda i: (i, 0)),
            pl.BlockSpec((_B, 1), lambda i: (i, 0)),
        ],
        out_shape=[
            jax.ShapeDtypeStruct((n, 1), jnp.float32),
            jax.ShapeDtypeStruct((n, 1), jnp.float32),
        ],
    )(h, y3)
    out = pl.pallas_call(
        _stage2,
        in_specs=[
            pl.BlockSpec(memory_space=pltpu.SMEM),
            pl.BlockSpec((n,), lambda: (0,)),
            pl.BlockSpec((n,), lambda: (0,)),
        ],
        out_specs=pl.BlockSpec((n,), lambda: (0,)),
        out_shape=jax.ShapeDtypeStruct((n,), jnp.float32),
    )(scale, a.reshape(n), s.reshape(n))
    return out
